# Initial kernel scaffold; baseline (speedup 1.0000x reference)
import functools

import jax
import jax.numpy as jnp
from jax import lax
from jax.experimental import pallas as pl
from jax.experimental.pallas import tpu as pltpu
from jax.experimental.pallas import tpu_sc as plsc

VOCAB = 8192
N = 8 * 2048
NC, NS = 2, 16
NW = NC * NS
B_PER_W = N // NW
K = 8


@functools.partial(
    pl.kernel,
    mesh=plsc.VectorSubcoreMesh(core_axis_name="c", subcore_axis_name="s"),
    out_type=jax.ShapeDtypeStruct((N, VOCAB), jnp.float32),
    scratch_types=[
        pltpu.SMEM((B_PER_W,), jnp.int32),
        pltpu.SemaphoreType.DMA,
    ],
)
def _gather(idx_hbm, table_hbm, out_hbm, idx_s, sem):
    wid = lax.axis_index("s") * NC + lax.axis_index("c")
    base = wid * B_PER_W
    pltpu.sync_copy(idx_hbm.at[pl.ds(base, B_PER_W)], idx_s)

    def chunk(c, _):
        start = base + c * K
        copies = []
        for k in range(K):
            row = idx_s[c * K + k]
            copies.append(
                pltpu.async_copy(
                    table_hbm.at[pl.ds(row, 1)],
                    out_hbm.at[pl.ds(start + k, 1)],
                    sem,
                )
            )
        for cp in copies:
            cp.wait()
        return 0

    lax.fori_loop(0, B_PER_W // K, chunk, 0)


def kernel(idx, table):
    b, t = idx.shape
    flat = _gather(idx.reshape(-1), table)
    return flat.reshape(b, t, VOCAB)


# speedup vs baseline: 2.0142x; 2.0142x over previous
import functools

import jax
import jax.numpy as jnp
from jax import lax
from jax.experimental import pallas as pl
from jax.experimental.pallas import tpu as pltpu
from jax.experimental.pallas import tpu_sc as plsc

VOCAB = 8192
N = 8 * 2048
NC, NS = 2, 16
NW = NC * NS
B_PER_W = N // NW
R = 4
NBUF = 2
NCH = B_PER_W // R


@functools.partial(
    pl.kernel,
    mesh=plsc.VectorSubcoreMesh(core_axis_name="c", subcore_axis_name="s"),
    out_type=jax.ShapeDtypeStruct((N, VOCAB), jnp.float32),
    scratch_types=[
        pltpu.VMEM((NCH, R), jnp.int32),
        pltpu.VMEM((NBUF, R, VOCAB), jnp.float32),
        pltpu.SemaphoreType.DMA,
        pltpu.SemaphoreType.DMA,
    ],
)
def _gather(idx_hbm, table_hbm, out_hbm, idx_v, buf, sem_g, sem_s):
    wid = lax.axis_index("s") * NC + lax.axis_index("c")
    row_base = wid * B_PER_W
    pltpu.sync_copy(idx_hbm.at[pl.ds(wid * NCH, NCH)], idx_v)

    def start_gather(c, s):
        pltpu.async_copy(table_hbm.at[idx_v.at[c]], buf.at[s], sem_g)

    def wait_gather(s):
        pltpu.make_async_copy(table_hbm.at[pl.ds(0, R)], buf.at[s], sem_g).wait()

    for s in range(NBUF):
        start_gather(s, s)

    def body(i, _):
        c0 = i * NBUF
        for s in range(NBUF):
            c = c0 + s
            wait_gather(s)
            scat = pltpu.async_copy(
                buf.at[s], out_hbm.at[pl.ds(row_base + c * R, R)], sem_s
            )
            scat.wait()

            @pl.when(c + NBUF < NCH)
            def _():
                start_gather(c + NBUF, s)

        return 0

    lax.fori_loop(0, NCH // NBUF, body, 0)


def kernel(idx, table):
    b, t = idx.shape
    flat = _gather(idx.reshape(N // R, R), table)
    return flat.reshape(b, t, VOCAB)
